# baseline (speedup 1.0000x reference)
import functools

import jax
import jax.numpy as jnp
from jax.experimental import pallas as pl
from jax.experimental.pallas import tpu as pltpu

STATE_DIM = 2048
ACTION_DIM = 512
BATCH = 512
FULL_K = STATE_DIM + ACTION_DIM

TILE = 512
N_TILES = STATE_DIM // TILE
NSLOTS_A = 2
NSLOTS_B = 4

_DNT = (((1,), (1,)), ((), ()))


def _dot_t(x, w):
    return jax.lax.dot_general(
        x, w, _DNT,
        precision=jax.lax.Precision.DEFAULT,
        preferred_element_type=jnp.float32,
    )


def _body(
    state_ref, action_ref, ns_hbm,
    w1_hbm, b1_ref, w2_hbm, b2_ref, wf_hbm, bf_ref,
    out_ref,
    xn_ref, h_ref, pn_ref, *scr,
):
    a_slots = scr[:NSLOTS_A]
    b_slots = scr[NSLOTS_A:NSLOTS_A + NSLOTS_B]
    a_sems = scr[NSLOTS_A + NSLOTS_B:2 * NSLOTS_A + NSLOTS_B]
    b_sems = scr[2 * NSLOTS_A + NSLOTS_B:2 * NSLOTS_A + 2 * NSLOTS_B]
    sem_n = scr[2 * NSLOTS_A + 2 * NSLOTS_B]

    def a_copy(t):
        return pltpu.make_async_copy(
            w1_hbm.at[pl.ds(t * TILE, TILE), :],
            a_slots[t % NSLOTS_A], a_sems[t % NSLOTS_A])

    def b_copy(i):
        t = i % N_TILES
        src = w2_hbm if i < N_TILES else wf_hbm
        return pltpu.make_async_copy(
            src.at[pl.ds(t * TILE, TILE), :],
            b_slots[i % NSLOTS_B], b_sems[i % NSLOTS_B])

    for t in range(NSLOTS_A):
        a_copy(t).start()
    for i in range(NSLOTS_B):
        b_copy(i).start()
    cp_n = pltpu.make_async_copy(ns_hbm, xn_ref, sem_n)
    cp_n.start()

    for t in range(N_TILES):
        a_copy(t).wait()
        w = a_slots[t % NSLOTS_A][...]
        col = pl.ds(t * TILE, TILE)
        acc = _dot_t(state_ref[...], w[:, :STATE_DIM])
        acc += _dot_t(action_ref[...], w[:, STATE_DIM:])
        h_ref[:, col] = jnp.maximum(acc + b1_ref[col][None, :], 0.0)
        if t + NSLOTS_A < N_TILES:
            a_copy(t + NSLOTS_A).start()

    for i in range(N_TILES):
        b_copy(i).wait()
        w = b_slots[i % NSLOTS_B][...]
        col = pl.ds(i * TILE, TILE)
        pn_ref[:, col] = _dot_t(h_ref[...], w) + b2_ref[col][None, :]
        b_copy(i + NSLOTS_B).start()

    sse = jnp.zeros((), jnp.float32)
    for i in range(N_TILES):
        b_copy(N_TILES + i).wait()
        if i == 0:
            cp_n.wait()
        w = b_slots[(N_TILES + i) % NSLOTS_B][...]
        col = pl.ds(i * TILE, TILE)
        b = bf_ref[col][None, :]
        fa = jnp.maximum(_dot_t(xn_ref[...], w) + b, 0.0)
        fp = jnp.maximum(_dot_t(pn_ref[...], w) + b, 0.0)
        d = fp - fa
        sse += jnp.sum(d * d)

    out_ref[...] = sse.reshape(1, 1)


@functools.partial(jax.jit, static_argnames=())
def kernel(state, action, next_state, W1, b1, W2, b2, Wf, bf):
    vmem = functools.partial(pl.BlockSpec, memory_space=pltpu.MemorySpace.VMEM)
    hbm = pl.BlockSpec(memory_space=pltpu.MemorySpace.HBM)
    sse = pl.pallas_call(
        _body,
        in_specs=[
            vmem(), vmem(), hbm,
            hbm, vmem(),
            hbm, vmem(),
            hbm, vmem(),
        ],
        out_specs=vmem(),
        out_shape=jax.ShapeDtypeStruct((1, 1), jnp.float32),
        scratch_shapes=(
            [pltpu.VMEM((BATCH, STATE_DIM), jnp.float32),
             pltpu.VMEM((BATCH, STATE_DIM), jnp.float32),
             pltpu.VMEM((BATCH, STATE_DIM), jnp.float32)]
            + [pltpu.VMEM((TILE, FULL_K), jnp.float32)
               for _ in range(NSLOTS_A)]
            + [pltpu.VMEM((TILE, STATE_DIM), jnp.float32)
               for _ in range(NSLOTS_B)]
            + [pltpu.SemaphoreType.DMA
               for _ in range(NSLOTS_A + NSLOTS_B + 1)]
        ),
    )(state, action, next_state, W1, b1, W2, b2, Wf, bf)

    pred_error = sse[0, 0] / jnp.float32(BATCH * STATE_DIM)
    novelty = jnp.float32(1.0)
    curiosity = pred_error * 0.5 + novelty * 0.5
    return jnp.stack([pred_error, novelty, curiosity])


# speedup vs baseline: 1.1618x; 1.1618x over previous
import functools

import jax
import jax.numpy as jnp
from jax.experimental import pallas as pl
from jax.experimental.pallas import tpu as pltpu

STATE_DIM = 2048
ACTION_DIM = 512
BATCH = 512
FULL_K = STATE_DIM + ACTION_DIM

TILE = 512
N_TILES = STATE_DIM // TILE
NSLOTS_B = 3

_DNT = (((1,), (1,)), ((), ()))


def _dot_t(x, w):
    return jax.lax.dot_general(
        x, w, _DNT,
        precision=jax.lax.Precision.DEFAULT,
        preferred_element_type=jnp.float32,
    )


def _body(
    state_ref, action_ref, ns_hbm,
    w1_hbm, b1_ref, w2_hbm, b2_ref, wf_hbm, bf_ref,
    out_ref,
    xn_ref, h_ref, pn_ref, *scr,
):
    a_bufs = scr[:N_TILES]
    b_slots = scr[N_TILES:N_TILES + NSLOTS_B]
    a_sems = scr[N_TILES + NSLOTS_B:2 * N_TILES + NSLOTS_B]
    b_sems = scr[2 * N_TILES + NSLOTS_B:2 * N_TILES + 2 * NSLOTS_B]
    sem_n = scr[2 * N_TILES + 2 * NSLOTS_B]

    def a_copy(t):
        return pltpu.make_async_copy(
            w1_hbm.at[pl.ds(t * TILE, TILE), :], a_bufs[t], a_sems[t])

    def b_copy(i):
        t = i % N_TILES
        src = w2_hbm if i < N_TILES else wf_hbm
        return pltpu.make_async_copy(
            src.at[pl.ds(t * TILE, TILE), :],
            b_slots[i % NSLOTS_B], b_sems[i % NSLOTS_B])

    for t in range(N_TILES):
        a_copy(t).start()
    for i in range(NSLOTS_B):
        b_copy(i).start()
    cp_n = pltpu.make_async_copy(ns_hbm, xn_ref, sem_n)
    cp_n.start()

    for t in range(N_TILES):
        a_copy(t).wait()
        w = a_bufs[t][...]
        col = pl.ds(t * TILE, TILE)
        acc = _dot_t(state_ref[...], w[:, :STATE_DIM])
        acc += _dot_t(action_ref[...], w[:, STATE_DIM:])
        h_ref[:, col] = jnp.maximum(acc + b1_ref[col][None, :], 0.0)

    for i in range(N_TILES):
        b_copy(i).wait()
        w = b_slots[i % NSLOTS_B][...]
        col = pl.ds(i * TILE, TILE)
        pn_ref[:, col] = _dot_t(h_ref[...], w) + b2_ref[col][None, :]
        if i + NSLOTS_B < 2 * N_TILES:
            b_copy(i + NSLOTS_B).start()

    sse = jnp.zeros((), jnp.float32)
    for i in range(N_TILES):
        b_copy(N_TILES + i).wait()
        if i == 0:
            cp_n.wait()
        w = b_slots[(N_TILES + i) % NSLOTS_B][...]
        col = pl.ds(i * TILE, TILE)
        if N_TILES + i + NSLOTS_B < 2 * N_TILES:
            b_copy(N_TILES + i + NSLOTS_B).start()
        b = bf_ref[col][None, :]
        fa = jnp.maximum(_dot_t(xn_ref[...], w) + b, 0.0)
        fp = jnp.maximum(_dot_t(pn_ref[...], w) + b, 0.0)
        d = fp - fa
        sse += jnp.sum(d * d)

    out_ref[...] = sse.reshape(1, 1)


@functools.partial(jax.jit, static_argnames=())
def kernel(state, action, next_state, W1, b1, W2, b2, Wf, bf):
    vmem = functools.partial(pl.BlockSpec, memory_space=pltpu.MemorySpace.VMEM)
    hbm = pl.BlockSpec(memory_space=pltpu.MemorySpace.HBM)
    sse = pl.pallas_call(
        _body,
        in_specs=[
            vmem(), vmem(), hbm,
            hbm, vmem(),
            hbm, vmem(),
            hbm, vmem(),
        ],
        out_specs=vmem(),
        out_shape=jax.ShapeDtypeStruct((1, 1), jnp.float32),
        scratch_shapes=(
            [pltpu.VMEM((BATCH, STATE_DIM), jnp.float32),
             pltpu.VMEM((BATCH, STATE_DIM), jnp.float32),
             pltpu.VMEM((BATCH, STATE_DIM), jnp.float32)]
            + [pltpu.VMEM((TILE, FULL_K), jnp.float32)
               for _ in range(N_TILES)]
            + [pltpu.VMEM((TILE, STATE_DIM), jnp.float32)
               for _ in range(NSLOTS_B)]
            + [pltpu.SemaphoreType.DMA
               for _ in range(N_TILES + NSLOTS_B + 1)]
        ),
    )(state, action, next_state, W1, b1, W2, b2, Wf, bf)

    pred_error = sse[0, 0] / jnp.float32(BATCH * STATE_DIM)
    novelty = jnp.float32(1.0)
    curiosity = pred_error * 0.5 + novelty * 0.5
    return jnp.stack([pred_error, novelty, curiosity])


# speedup vs baseline: 1.1629x; 1.0010x over previous
import functools

import jax
import jax.numpy as jnp
from jax.experimental import pallas as pl
from jax.experimental.pallas import tpu as pltpu

STATE_DIM = 2048
ACTION_DIM = 512
BATCH = 512
FULL_K = STATE_DIM + ACTION_DIM

W1_ROWS = 1024
N_W1 = STATE_DIM // W1_ROWS
BROWS = 1024

_B_TILES = [
    ("w2", 0, 1024),
    ("w2", 1024, 1024),
    ("wf", 0, 1024),
    ("wf", 1024, 512),
    ("wf", 1536, 512),
]
NSLOTS_B = 2

_DNT = (((1,), (1,)), ((), ()))


def _dot_t(x, w):
    return jax.lax.dot_general(
        x, w, _DNT,
        precision=jax.lax.Precision.DEFAULT,
        preferred_element_type=jnp.float32,
    )


def _body(
    state_ref, action_ref, ns_hbm,
    w1_hbm, b1_ref, w2_hbm, b2_ref, wf_hbm, bf_ref,
    out_ref,
    xn_ref, h_ref, pn_ref, *scr,
):
    a_bufs = scr[:N_W1]
    b_slots = scr[N_W1:N_W1 + NSLOTS_B]
    a_sems = scr[N_W1 + NSLOTS_B:2 * N_W1 + NSLOTS_B]
    b_sems = scr[2 * N_W1 + NSLOTS_B:2 * N_W1 + 2 * NSLOTS_B]
    sem_n = scr[2 * N_W1 + 2 * NSLOTS_B]

    def a_copy(t):
        return pltpu.make_async_copy(
            w1_hbm.at[pl.ds(t * W1_ROWS, W1_ROWS), :], a_bufs[t], a_sems[t])

    def b_copy(i):
        kind, r0, rows = _B_TILES[i]
        src = w2_hbm if kind == "w2" else wf_hbm
        slot = i % NSLOTS_B
        return pltpu.make_async_copy(
            src.at[pl.ds(r0, rows), :],
            b_slots[slot].at[pl.ds(0, rows), :], b_sems[slot])

    for t in range(N_W1):
        a_copy(t).start()
    for i in range(NSLOTS_B):
        b_copy(i).start()
    cp_n = pltpu.make_async_copy(ns_hbm, xn_ref, sem_n)
    cp_n.start()

    for t in range(N_W1):
        a_copy(t).wait()
        w = a_bufs[t][...]
        col = pl.ds(t * W1_ROWS, W1_ROWS)
        acc = _dot_t(state_ref[...], w[:, :STATE_DIM])
        acc += _dot_t(action_ref[...], w[:, STATE_DIM:])
        h_ref[:, col] = jnp.maximum(acc + b1_ref[col][None, :], 0.0)

    sse = jnp.zeros((), jnp.float32)
    for i, (kind, r0, rows) in enumerate(_B_TILES):
        b_copy(i).wait()
        if kind == "wf" and r0 == 0:
            cp_n.wait()
        w = b_slots[i % NSLOTS_B][pl.ds(0, rows), :]
        col = pl.ds(r0, rows)
        if i + NSLOTS_B < len(_B_TILES):
            b_copy(i + NSLOTS_B).start()
        if kind == "w2":
            pn_ref[:, col] = _dot_t(h_ref[...], w) + b2_ref[col][None, :]
        else:
            b = bf_ref[col][None, :]
            fa = jnp.maximum(_dot_t(xn_ref[...], w) + b, 0.0)
            fp = jnp.maximum(_dot_t(pn_ref[...], w) + b, 0.0)
            d = fp - fa
            sse += jnp.sum(d * d)

    out_ref[...] = sse.reshape(1, 1)


@functools.partial(jax.jit, static_argnames=())
def kernel(state, action, next_state, W1, b1, W2, b2, Wf, bf):
    vmem = functools.partial(pl.BlockSpec, memory_space=pltpu.MemorySpace.VMEM)
    hbm = pl.BlockSpec(memory_space=pltpu.MemorySpace.HBM)
    sse = pl.pallas_call(
        _body,
        in_specs=[
            vmem(), vmem(), hbm,
            hbm, vmem(),
            hbm, vmem(),
            hbm, vmem(),
        ],
        out_specs=vmem(),
        out_shape=jax.ShapeDtypeStruct((1, 1), jnp.float32),
        scratch_shapes=(
            [pltpu.VMEM((BATCH, STATE_DIM), jnp.float32),
             pltpu.VMEM((BATCH, STATE_DIM), jnp.float32),
             pltpu.VMEM((BATCH, STATE_DIM), jnp.float32)]
            + [pltpu.VMEM((W1_ROWS, FULL_K), jnp.float32)
               for _ in range(N_W1)]
            + [pltpu.VMEM((BROWS, STATE_DIM), jnp.float32)
               for _ in range(NSLOTS_B)]
            + [pltpu.SemaphoreType.DMA
               for _ in range(N_W1 + NSLOTS_B + 1)]
        ),
    )(state, action, next_state, W1, b1, W2, b2, Wf, bf)

    pred_error = sse[0, 0] / jnp.float32(BATCH * STATE_DIM)
    novelty = jnp.float32(1.0)
    curiosity = pred_error * 0.5 + novelty * 0.5
    return jnp.stack([pred_error, novelty, curiosity])
